# Initial kernel scaffold; baseline (speedup 1.0000x reference)
import jax
import jax.numpy as jnp
from jax.experimental import pallas as pl


def kernel(sparse_fea, dense_fea, sparse_coor, dense_coor, params):
    raise NotImplementedError("write your pallas kernel here")


# speedup vs baseline: 12.2486x; 12.2486x over previous
import functools

import jax
import jax.numpy as jnp
from jax import lax
from jax.experimental import pallas as pl
from jax.experimental.pallas import tpu as pltpu
from jax.experimental.pallas import tpu_sc as plsc

F32 = jnp.float32
HI = jax.lax.Precision.HIGHEST

B = 8
S = 32
P = 32
L = S * P
C_SP = 128
C_DN = 128
SU = 256
K_SP = 2
K_DN = 10
DW = 128


def _silu(x):
    return x * jax.nn.sigmoid(x)


def _bn_rows(x, g, b):
    m = jnp.mean(x, axis=0, keepdims=True)
    v = jnp.mean((x - m) * (x - m), axis=0, keepdims=True)
    return (x - m) / jnp.sqrt(v + 1e-5) * g + b


def _roll_axis1(x3, off):
    n = x3.shape[1]
    off = off % n
    if off == 0:
        return x3
    return jnp.concatenate([x3[:, off:, :], x3[:, :off, :]], axis=1)


def _dot(a, b):
    return jnp.dot(a, b, precision=HI, preferred_element_type=F32)


def _pc1a_body(xd_ref, w5_ref, cb_ref, out_ref):
    xd3 = xd_ref[...]
    acc = cb_ref[...] + jnp.zeros((L, C_DN), F32)
    for t in range(5):
        sh = _roll_axis1(xd3, t - 2).reshape(L, C_DN)
        acc = acc + _dot(sh, w5_ref[t])
    out_ref[...] = acc.reshape(1, L, C_DN)


def _pc1b_body(cr_ref, xs_ref, csf_ref, csl_ref,
               cbng_ref, cbnb_ref,
               mwT_ref, mb_ref, mbng_ref, mbnb_ref,
               suWc_ref, suWg_ref, suW3_ref, suWa_ref, su_bm_ref,
               subng_ref, subnb_ref,
               sp_out_ref):
    df = _silu(_bn_rows(cr_ref[...], cbng_ref[...], cbnb_ref[...]))
    spd = jnp.max(df.reshape(B * S, P, C_DN), axis=1)
    xs_f = xs_ref[...]
    us0 = jnp.concatenate([xs_f, spd], axis=1)
    us = _dot(us0, mwT_ref[...]) + mb_ref[...]
    us = _silu(_bn_rows(us, mbng_ref[...], mbnb_ref[...]))
    csf = csf_ref[...]
    cw3s = _dot(csf, suW3_ref[...])
    cen_s = _dot(us, suWc_ref[...]) + su_bm_ref[...] - cw3s
    srcS = jnp.concatenate(
        [_dot(us, suWg_ref[...]) + cw3s, _dot(csf, suWa_ref[...])], axis=1)
    csn3 = csf.reshape(B, S, 3)
    srcS3 = srcS.reshape(B, S, C_SP + 1)
    cen_s3 = cen_s.reshape(B, S, C_SP)
    ilane = lax.broadcasted_iota(jnp.int32, (S, S), 1)
    outs = []
    for b in range(8):
        cb3 = csn3[b]
        clb = csl_ref[b]
        sq_c = jnp.sum(cb3 * cb3, axis=1, keepdims=True)
        sq_r = jnp.sum(clb * clb, axis=0, keepdims=True)
        cross = (cb3[:, 0:1] * clb[0:1, :] + cb3[:, 1:2] * clb[1:2, :]
                 + cb3[:, 2:3] * clb[2:3, :])
        d2 = sq_c + sq_r - 2.0 * cross
        msgs, logits = [], []
        for _ in range(K_SP):
            mrow = jnp.min(d2, axis=1, keepdims=True)
            idxc = jnp.min(jnp.where(d2 <= mrow, ilane, 2 * S),
                           axis=1, keepdims=True)
            oh = (ilane == idxc).astype(F32)
            g = _dot(oh, srcS3[b])
            logits.append(g[:, C_SP:C_SP + 1])
            msgs.append(_silu(cen_s3[b] + g[:, :C_SP]))
            d2 = jnp.where(ilane == idxc, F32(1e30), d2)
        mx = jnp.maximum(logits[0], logits[1])
        e0 = jnp.exp(logits[0] - mx)
        e1 = jnp.exp(logits[1] - mx)
        outs.append((e0 * msgs[0] + e1 * msgs[1]) / (e0 + e1))
    sp_raw = jnp.concatenate(outs, axis=0)
    sp_out_ref[...] = _silu(_bn_rows(sp_raw, subng_ref[...], subnb_ref[...]))


def _pc1c_body(xd_ref, xs_ref, stdTa_ref, stdTb_ref, stdb_ref, udraw_ref):
    xd_f = xd_ref[...].reshape(L, C_DN)
    rep = jnp.broadcast_to(
        xs_ref[...].reshape(S, 1, C_SP), (S, P, C_SP)).reshape(L, C_SP)
    ud = _dot(xd_f, stdTa_ref[...]) + _dot(rep, stdTb_ref[...]) + stdb_ref[...]
    udraw_ref[...] = ud.reshape(1, L, SU)


def _bnstats_body(x_ref, m_ref, v_ref):
    x = x_ref[...]
    m = jnp.mean(x, axis=0, keepdims=True)
    m_ref[...] = m
    v_ref[...] = jnp.mean((x - m) * (x - m), axis=0, keepdims=True)


def _pc1d_body(udraw_ref, m_ref, v_ref, stdg_ref, stdbb_ref, cdf_ref,
               duWc_ref, duWg_ref, duW3_ref, duWa_ref, du_bm_ref,
               cen_ref, src_ref, cwa_ref):
    udr = udraw_ref[...].reshape(L, SU)
    ud = _silu((udr - m_ref[...]) / jnp.sqrt(v_ref[...] + 1e-5)
               * stdg_ref[...] + stdbb_ref[...])
    cdf = cdf_ref[...].reshape(L, 3)
    cw3d = _dot(cdf, duW3_ref[...])
    cen_ref[...] = (_dot(ud, duWc_ref[...]) + du_bm_ref[...]
                    - cw3d).reshape(1, L, C_DN)
    src_ref[...] = (_dot(ud, duWg_ref[...]) + cw3d).reshape(1, L, DW)
    cwa_ref[...] = jnp.broadcast_to(
        _dot(cdf, duWa_ref[...]), (L, 8)).reshape(1, L, 8)


def _pc2_body(cdn_ref, cdl_ref, cwa_ref, idx_ref, lg_ref):
    b = pl.program_id(0)
    cdn = cdn_ref[0]
    cdl = cdl_ref[0]
    cwa_row = cwa_ref[0][0:1, :]
    sq_c = jnp.sum(cdn * cdn, axis=1, keepdims=True)
    sq_r = jnp.sum(cdl * cdl, axis=0, keepdims=True)
    cross = (cdn[:, 0:1] * cdl[0:1, :] + cdn[:, 1:2] * cdl[1:2, :]
             + cdn[:, 2:3] * cdl[2:3, :])
    d2 = sq_c + sq_r - 2.0 * cross
    ilane = lax.broadcasted_iota(jnp.int32, (L, L), 1)
    for j in range(K_DN):
        mrow = jnp.min(d2, axis=1, keepdims=True)
        idxc = jnp.min(jnp.where(d2 <= mrow, ilane, 2 * L),
                       axis=1, keepdims=True)
        sel = ilane == idxc
        idx_ref[0, :, j:j + 1] = idxc + b * L
        lg_ref[0, :, j:j + 1] = jnp.sum(
            jnp.where(sel, cwa_row, F32(0.0)), axis=1, keepdims=True)
        d2 = jnp.where(sel, F32(1e30), d2)
    idx_ref[0, :, K_DN:16] = jnp.zeros((L, 16 - K_DN), jnp.int32)
    lg_ref[0, :, K_DN:16] = jnp.zeros((L, 16 - K_DN), F32)


NC, NS = 2, 16
NW = NC * NS
ROWS = B * K_DN * L
RPW = ROWS // NW
CH = 256
NCHUNK = RPW // CH


def _sc_gather(table, idx):
    mesh = plsc.VectorSubcoreMesh(core_axis_name="c", subcore_axis_name="s")

    @functools.partial(
        pl.kernel, mesh=mesh,
        out_type=jax.ShapeDtypeStruct((ROWS, DW), F32),
        scratch_types=[
            pltpu.VMEM((RPW,), jnp.int32),
            pltpu.VMEM((CH, DW), F32),
            pltpu.SemaphoreType.DMA,
        ],
    )
    def k(table_hbm, idx_hbm, out_hbm, idx_v, rows_v, sem):
        wid = lax.axis_index("s") * NC + lax.axis_index("c")
        base = wid * RPW
        pltpu.sync_copy(idx_hbm.at[pl.ds(base, RPW)], idx_v)
        for i in range(NCHUNK):
            pltpu.async_copy(
                table_hbm.at[idx_v.at[pl.ds(i * CH, CH)]], rows_v, sem).wait()
            pltpu.sync_copy(rows_v, out_hbm.at[pl.ds(base + i * CH, CH)])

    return k(table, idx)


def _pc3a_body(g_ref, cen_ref, lg_ref, out_ref):
    gb = g_ref[0]
    cen = cen_ref[0]
    lmat = lg_ref[0][:, :K_DN]
    mx = jnp.max(lmat, axis=1, keepdims=True)
    e = jnp.exp(lmat - mx)
    s = jnp.sum(e, axis=1, keepdims=True)
    acc = jnp.zeros((L, C_DN), F32)
    for k in range(K_DN):
        acc = acc + e[:, k:k + 1] * _silu(cen + gb[k, :, :C_DN])
    out_ref[0] = acc / s


def _pc3b_body(x_ref, g_ref, b_ref, out_ref):
    out_ref[...] = _silu(_bn_rows(x_ref[...], g_ref[...], b_ref[...]))


def _pc3c_body(xe_ref, xo_ref, w6_ref, cb_ref, g_ref, b_ref, out_ref):
    xe = xe_ref[...]
    xo = xo_ref[...]
    H = L // 2
    acc = _dot(_roll_axis1(xe, -1).reshape(B * H, C_DN), w6_ref[0])
    acc = acc + _dot(_roll_axis1(xo, -1).reshape(B * H, C_DN), w6_ref[1])
    acc = acc + _dot(xe.reshape(B * H, C_DN), w6_ref[2])
    acc = acc + _dot(xo.reshape(B * H, C_DN), w6_ref[3])
    acc = acc + _dot(_roll_axis1(xe, 1).reshape(B * H, C_DN), w6_ref[4])
    acc = acc + _dot(_roll_axis1(xo, 1).reshape(B * H, C_DN), w6_ref[5])
    acc = acc + cb_ref[...]
    out_ref[...] = _silu(_bn_rows(acc, g_ref[...], b_ref[...]))


def _row(v):
    return v.reshape(1, -1)


def kernel(sparse_fea, dense_fea, sparse_coor, dense_coor, params):
    p = params
    xd = jnp.transpose(dense_fea.reshape(B, C_DN, L), (0, 2, 1))
    xs = jnp.transpose(sparse_fea, (0, 2, 1))
    cdf = jnp.transpose(dense_coor.reshape(B, 3, L), (0, 2, 1))
    cdl = jnp.concatenate(
        [dense_coor.reshape(B, 3, L), jnp.zeros((B, 5, L), F32)], axis=1)
    csf = jnp.transpose(sparse_coor, (0, 2, 1)).reshape(B * S, 3)
    csl = jnp.concatenate(
        [sparse_coor, jnp.zeros((B, 5, S), F32)], axis=1)

    w5 = jnp.transpose(p['dts_cw'], (2, 1, 0))
    w6 = jnp.transpose(p['ds_cw'], (2, 1, 0))
    suWc = p['su_wm'][:SU] - p['su_wm'][SU:2 * SU]
    suWg = p['su_wm'][SU:2 * SU]
    suW3 = p['su_wm'][2 * SU:]
    duWc = p['du_wm'][:SU] - p['du_wm'][SU:2 * SU]
    duWg = p['du_wm'][SU:2 * SU]
    duW3 = p['du_wm'][2 * SU:]

    conv_raw = pl.pallas_call(
        _pc1a_body,
        grid=(B,),
        in_specs=[
            pl.BlockSpec((1, L, C_DN), lambda b: (b, 0, 0)),
            pl.BlockSpec((5, C_DN, C_DN), lambda b: (0, 0, 0)),
            pl.BlockSpec((1, C_DN), lambda b: (0, 0)),
        ],
        out_specs=pl.BlockSpec((1, L, C_DN), lambda b: (b, 0, 0)),
        out_shape=jax.ShapeDtypeStruct((B, L, C_DN), F32),
    )(xd, w5, _row(p['dts_cb']))
    conv_raw = conv_raw.reshape(B * L, C_DN)

    sp_out_f = pl.pallas_call(
        _pc1b_body,
        out_shape=jax.ShapeDtypeStruct((B * S, C_SP), F32),
    )(conv_raw, xs.reshape(B * S, C_SP), csf, csl,
      _row(p['dts_bng']), _row(p['dts_bnb']),
      p['dts_mw'].T, _row(p['dts_mb']), _row(p['dts_mbng']), _row(p['dts_mbnb']),
      suWc, suWg, suW3, p['su_wa'], _row(p['su_bm']),
      _row(p['su_bng']), _row(p['su_bnb']))

    udraw = pl.pallas_call(
        _pc1c_body,
        grid=(B,),
        in_specs=[
            pl.BlockSpec((1, L, C_DN), lambda b: (b, 0, 0)),
            pl.BlockSpec((1, S, C_SP), lambda b: (b, 0, 0)),
            pl.BlockSpec((C_DN, SU), lambda b: (0, 0)),
            pl.BlockSpec((C_SP, SU), lambda b: (0, 0)),
            pl.BlockSpec((1, SU), lambda b: (0, 0)),
        ],
        out_specs=pl.BlockSpec((1, L, SU), lambda b: (b, 0, 0)),
        out_shape=jax.ShapeDtypeStruct((B, L, SU), F32),
    )(xd, xs, p['std_mw'].T[:C_DN], p['std_mw'].T[C_DN:], _row(p['std_mb']))

    ud_m, ud_v = pl.pallas_call(
        _bnstats_body,
        out_shape=[
            jax.ShapeDtypeStruct((1, SU), F32),
            jax.ShapeDtypeStruct((1, SU), F32),
        ],
    )(udraw.reshape(B * L, SU))

    cen3, src3, cwa3 = pl.pallas_call(
        _pc1d_body,
        grid=(B,),
        in_specs=[
            pl.BlockSpec((1, L, SU), lambda b: (b, 0, 0)),
            pl.BlockSpec((1, SU), lambda b: (0, 0)),
            pl.BlockSpec((1, SU), lambda b: (0, 0)),
            pl.BlockSpec((1, SU), lambda b: (0, 0)),
            pl.BlockSpec((1, SU), lambda b: (0, 0)),
            pl.BlockSpec((1, L, 3), lambda b: (b, 0, 0)),
            pl.BlockSpec((SU, C_DN), lambda b: (0, 0)),
            pl.BlockSpec((SU, C_DN), lambda b: (0, 0)),
            pl.BlockSpec((3, C_DN), lambda b: (0, 0)),
            pl.BlockSpec((3, 1), lambda b: (0, 0)),
            pl.BlockSpec((1, C_DN), lambda b: (0, 0)),
        ],
        out_specs=[
            pl.BlockSpec((1, L, C_DN), lambda b: (b, 0, 0)),
            pl.BlockSpec((1, L, DW), lambda b: (b, 0, 0)),
            pl.BlockSpec((1, L, 8), lambda b: (b, 0, 0)),
        ],
        out_shape=[
            jax.ShapeDtypeStruct((B, L, C_DN), F32),
            jax.ShapeDtypeStruct((B, L, DW), F32),
            jax.ShapeDtypeStruct((B, L, 8), F32),
        ],
    )(udraw, ud_m, ud_v, _row(p['std_bng']), _row(p['std_bnb']), cdf,
      duWc, duWg, duW3, p['du_wa'], _row(p['du_bm']))
    cen_f = cen3.reshape(B * L, C_DN)
    src_f = src3.reshape(B * L, DW)
    cwa_f = cwa3.reshape(B * L, 8)

    cwaR = jnp.broadcast_to(
        cwa_f[:, 0].reshape(B, 1, L), (B, 8, L))
    idxg, lgT = pl.pallas_call(
        _pc2_body,
        grid=(B,),
        in_specs=[
            pl.BlockSpec((1, L, 3), lambda b: (b, 0, 0)),
            pl.BlockSpec((1, 8, L), lambda b: (b, 0, 0)),
            pl.BlockSpec((1, 8, L), lambda b: (b, 0, 0)),
        ],
        out_specs=[
            pl.BlockSpec((1, L, 16), lambda b: (b, 0, 0)),
            pl.BlockSpec((1, L, 16), lambda b: (b, 0, 0)),
        ],
        out_shape=[
            jax.ShapeDtypeStruct((B, L, 16), jnp.int32),
            jax.ShapeDtypeStruct((B, L, 16), F32),
        ],
    )(cdf, cdl, cwaR)

    idx_flat = jnp.transpose(idxg[:, :, :K_DN], (0, 2, 1)).reshape(ROWS)
    g = _sc_gather(src_f, idx_flat)

    o3 = pl.pallas_call(
        _pc3a_body,
        grid=(B,),
        in_specs=[
            pl.BlockSpec((1, K_DN, L, DW), lambda b: (b, 0, 0, 0)),
            pl.BlockSpec((1, L, C_DN), lambda b: (b, 0, 0)),
            pl.BlockSpec((1, L, 16), lambda b: (b, 0, 0)),
        ],
        out_specs=pl.BlockSpec((1, L, C_DN), lambda b: (b, 0, 0)),
        out_shape=jax.ShapeDtypeStruct((B, L, C_DN), F32),
    )(g.reshape(B, K_DN, L, DW), cen_f.reshape(B, L, C_DN), lgT)

    dnf = pl.pallas_call(
        _pc3b_body,
        out_shape=jax.ShapeDtypeStruct((B * L, C_DN), F32),
    )(o3.reshape(B * L, C_DN), _row(p['du_bng']), _row(p['du_bnb']))

    dnf3 = dnf.reshape(B, L, C_DN)
    xe = dnf3[:, 0::2, :]
    xo = dnf3[:, 1::2, :]
    y = pl.pallas_call(
        _pc3c_body,
        out_shape=jax.ShapeDtypeStruct((B * L // 2, C_DN), F32),
    )(xe, xo, w6, _row(p['ds_cb']), _row(p['ds_bng']), _row(p['ds_bnb']))

    sp_out = jnp.transpose(sp_out_f.reshape(B, S, C_SP), (0, 2, 1))
    x = jnp.transpose(y.reshape(B, L // 2, C_DN), (0, 2, 1)).reshape(
        B, C_DN, S, P // 2)
    coor_ds = dense_coor[..., ::2]
    return (sp_out, x, coor_ds)
